# baseline (speedup 1.0000x reference)
import functools

import jax
import jax.numpy as jnp
from jax import lax
from jax.experimental import pallas as pl
from jax.experimental.pallas import tpu as pltpu
from jax.experimental.pallas import tpu_sc as plsc

BATCH = 16384
EMB = 32
NC = 2
NS = 16
NW = NC * NS
B_PER_W = BATCH // NW


def _sc_gather_body(uid_hbm, iid_hbm, uemb_hbm, iemb_hbm, ubias_hbm,
                    ibias_hbm, eu_out, ev_out, ub_out, ib_out,
                    uidx_v, iidx_v, eu_v, ev_v, ub_v, ib_v,
                    sem_eu, sem_ev, sem_ub, sem_ib):
    wid = lax.axis_index("s") * NC + lax.axis_index("c")
    base = wid * B_PER_W
    sl = pl.ds(base, B_PER_W)
    pltpu.sync_copy(uid_hbm.at[sl], uidx_v)
    pltpu.sync_copy(iid_hbm.at[sl], iidx_v)
    cp_eu = pltpu.async_copy(uemb_hbm.at[uidx_v], eu_v, sem_eu)
    cp_ev = pltpu.async_copy(iemb_hbm.at[iidx_v], ev_v, sem_ev)
    cp_ub = pltpu.async_copy(ubias_hbm.at[uidx_v], ub_v, sem_ub)
    cp_ib = pltpu.async_copy(ibias_hbm.at[iidx_v], ib_v, sem_ib)
    cp_eu.wait()
    cp_ev.wait()
    cp_ub.wait()
    cp_ib.wait()
    pltpu.sync_copy(eu_v, eu_out.at[sl])
    pltpu.sync_copy(ev_v, ev_out.at[sl])
    pltpu.sync_copy(ub_v, ub_out.at[sl])
    pltpu.sync_copy(ib_v, ib_out.at[sl])


@jax.jit
def _sc_gather(uid, iid, user_emb, item_emb, ubias, ibias):
    mesh = plsc.VectorSubcoreMesh(core_axis_name="c", subcore_axis_name="s",
                                  num_cores=NC, num_subcores=NS)
    return pl.kernel(
        _sc_gather_body,
        out_type=(
            jax.ShapeDtypeStruct((BATCH, EMB), jnp.float32),
            jax.ShapeDtypeStruct((BATCH, EMB), jnp.float32),
            jax.ShapeDtypeStruct((BATCH,), jnp.float32),
            jax.ShapeDtypeStruct((BATCH,), jnp.float32),
        ),
        mesh=mesh,
        scratch_types=[
            pltpu.VMEM((B_PER_W,), jnp.int32),
            pltpu.VMEM((B_PER_W,), jnp.int32),
            pltpu.VMEM((B_PER_W, EMB), jnp.float32),
            pltpu.VMEM((B_PER_W, EMB), jnp.float32),
            pltpu.VMEM((B_PER_W,), jnp.float32),
            pltpu.VMEM((B_PER_W,), jnp.float32),
            pltpu.SemaphoreType.DMA,
            pltpu.SemaphoreType.DMA,
            pltpu.SemaphoreType.DMA,
            pltpu.SemaphoreType.DMA,
        ],
        compiler_params=pltpu.CompilerParams(use_tc_tiling_on_sc=False),
        name="recsys_sc_gather",
    )(uid, iid, user_emb, item_emb, ubias, ibias)


def _tc_dense_body(eu_ref, ev_ref, uw_ref, ub_ref, iw_ref, ib_ref,
                   ubias_ref, ibias_ref, out_ref):
    uw = uw_ref[...]
    iw = iw_ref[...]
    ub = ub_ref[...]
    ib = ib_ref[...]
    f32 = jnp.float32
    eu = eu_ref[...]
    ev = ev_ref[...]
    u1 = jnp.dot(eu, uw, preferred_element_type=f32) + ub
    u2 = jnp.dot(u1, uw, preferred_element_type=f32) + ub
    v1 = jnp.dot(ev, iw, preferred_element_type=f32) + ib
    v2 = jnp.dot(v1, iw, preferred_element_type=f32) + ib
    s = jnp.sum(u2 * v2)
    x = s + ubias_ref[...] + ibias_ref[...]
    out_ref[...] = jax.nn.sigmoid(x)


@jax.jit
def _tc_dense(eu, ev, user_W, user_b, item_W, item_b, ubias, ibias):
    vmem = functools.partial(pl.BlockSpec, memory_space=pltpu.VMEM)
    return pl.pallas_call(
        _tc_dense_body,
        out_shape=jax.ShapeDtypeStruct((128, 128), jnp.float32),
        in_specs=[vmem()] * 8,
        out_specs=vmem(),
        name="recsys_tc_dense",
    )(eu, ev, user_W, user_b, item_W, item_b,
      ubias.reshape(128, 128), ibias.reshape(128, 128))


def kernel(inputs, user_emb, user_W, user_b, user_bias_tab, item_emb,
           item_W, item_b, item_bias_tab):
    uid = inputs[:, 0]
    iid = inputs[:, 1]
    eu, ev, ub, ib = _sc_gather(uid, iid, user_emb, item_emb,
                                user_bias_tab.reshape(-1),
                                item_bias_tab.reshape(-1))
    out = _tc_dense(eu, ev, user_W, user_b, item_W, item_b, ub, ib)
    return out.reshape(BATCH, 1)
